# baseline (speedup 1.0000x reference)
import functools

import jax
import jax.numpy as jnp
from jax import lax
from jax.experimental import pallas as pl
from jax.experimental.pallas import tpu as pltpu
from jax.experimental.pallas import tpu_sc as plsc

B = 16384
L = 200
VOCAB = 1000
VPAD = 1024
EMB = 32
H1 = 16

LPAD = 201
NW = 32
BAGS_PER_W = B // NW
GROUPS = BAGS_PER_W // 16


def _ptable_body(t_ref, w_ref, o_ref):
    o_ref[...] = jnp.dot(
        t_ref[...], w_ref[...], preferred_element_type=jnp.float32
    ) * (1.0 / L)


def _make_ptable(table_pad, w1t):
    return pl.pallas_call(
        _ptable_body,
        out_shape=jax.ShapeDtypeStruct((VPAD, H1), jnp.float32),
    )(table_pad, w1t)


def _full(v):
    return jnp.full((16,), v, dtype=jnp.int32)


def _sc_body(ptable_hbm, x_hbm, wpack_hbm, out_hbm,
             ptable_v, x_v, out_v, wp_v):
    wid = lax.axis_index("c") * 16 + lax.axis_index("s")
    base = wid * BAGS_PER_W

    pltpu.sync_copy(ptable_hbm, ptable_v)
    pltpu.sync_copy(x_hbm.at[pl.ds(base, BAGS_PER_W)], x_v.at[:, pl.ds(0, L)])
    pltpu.sync_copy(wpack_hbm, wp_v)

    lane = lax.iota(jnp.int32, 16)
    eoffs = tuple((lane + c) & 15 for c in range(H1))

    def group_body(g, carry):
        rows = lane + g * 16

        zero_accs = tuple(jnp.zeros((16,), jnp.float32) for _ in range(H1))

        @plsc.parallel_loop(0, L, unroll=8, carry=zero_accs)
        def accs(l, accs):
            lvec = jnp.broadcast_to(l, (16,))
            ivec = plsc.load_gather(x_v, [rows, lvec])
            return tuple(
                accs[c] + plsc.load_gather(ptable_v, [ivec, eoffs[c]])
                for c in range(H1)
            )

        z = plsc.load_gather(wp_v, [_full(40)])
        for c in range(H1):
            b1rot = plsc.load_gather(wp_v, [eoffs[c] + 8])
            w2rot = plsc.load_gather(wp_v, [eoffs[c] + 24])
            z = z + w2rot * jnp.maximum(accs[c] + b1rot, 0.0)
        out_v[pl.ds(g * 16, 16)] = 1.0 / (1.0 + jnp.exp(-z))
        return carry

    lax.fori_loop(0, GROUPS, group_body, 0)
    pltpu.sync_copy(out_v, out_hbm.at[pl.ds(base, BAGS_PER_W)])


_sc_call = functools.partial(
    pl.kernel,
    out_type=jax.ShapeDtypeStruct((B,), jnp.float32),
    mesh=plsc.VectorSubcoreMesh(core_axis_name="c", subcore_axis_name="s"),
    compiler_params=pltpu.CompilerParams(
        use_tc_tiling_on_sc=False,
        needs_layout_passes=False,
        disable_bounds_checks=True,
    ),
    scratch_types=[
        pltpu.VMEM((VPAD, H1), jnp.float32),
        pltpu.VMEM((BAGS_PER_W, LPAD), jnp.int32),
        pltpu.VMEM((BAGS_PER_W,), jnp.float32),
        pltpu.VMEM((48,), jnp.float32),
    ],
)(_sc_body)


def kernel(x, table, W1, b1, W2, b2):
    table_pad = jnp.pad(table, ((0, VPAD - VOCAB), (0, 0)))
    ptable = _make_ptable(table_pad, W1.T)
    wpack = jnp.concatenate([
        jnp.zeros((8,), jnp.float32), b1, W2.reshape(H1),
        jnp.broadcast_to(b2, (8,)),
    ]).astype(jnp.float32)
    out = _sc_call(ptable, x, wpack)
    return out.reshape(B, 1)


# speedup vs baseline: 1.1813x; 1.1813x over previous
import functools

import jax
import jax.numpy as jnp
from jax import lax
from jax.experimental import pallas as pl
from jax.experimental.pallas import tpu as pltpu
from jax.experimental.pallas import tpu_sc as plsc

B = 16384
L = 200
VOCAB = 1000
VPAD = 1024
EMB = 32
H1 = 16

LPAD = 201
NW = 32
BAGS_PER_W = B // NW
GROUPS = BAGS_PER_W // 16


def _ptable_body(t_ref, w_ref, o_ref):
    o_ref[...] = jnp.dot(
        t_ref[...], w_ref[...], preferred_element_type=jnp.float32
    ) * (1.0 / L)


def _make_ptable(table_pad, w1t):
    return pl.pallas_call(
        _ptable_body,
        out_shape=jax.ShapeDtypeStruct((VPAD, H1), jnp.float32),
    )(table_pad, w1t)


def _full(v):
    return jnp.full((16,), v, dtype=jnp.int32)


def _sc_body(ptable_hbm, x_hbm, wpack_hbm, out_hbm,
             ptable_v, x_v, out_v, wp_v):
    wid = lax.axis_index("c") * 16 + lax.axis_index("s")
    base = wid * BAGS_PER_W

    pltpu.sync_copy(ptable_hbm, ptable_v)
    pltpu.sync_copy(x_hbm.at[pl.ds(base, BAGS_PER_W)], x_v.at[:, pl.ds(0, L)])
    pltpu.sync_copy(wpack_hbm, wp_v)

    lane = lax.iota(jnp.int32, 16)
    eoffs = tuple((lane + c) & 15 for c in range(H1))

    def group_body(g, carry):
        rows = lane + g * 16

        zero_accs = tuple(jnp.zeros((16,), jnp.float32) for _ in range(H1))

        @plsc.parallel_loop(0, L, unroll=4, carry=zero_accs)
        def accs(l, accs):
            lvec = jnp.broadcast_to(l, (16,))
            ivec = plsc.load_gather(x_v, [rows, lvec])
            return tuple(
                accs[c] + plsc.load_gather(ptable_v, [ivec, eoffs[c]])
                for c in range(H1)
            )

        z = plsc.load_gather(wp_v, [_full(40)])
        for c in range(H1):
            b1rot = plsc.load_gather(wp_v, [eoffs[c] + 8])
            w2rot = plsc.load_gather(wp_v, [eoffs[c] + 24])
            z = z + w2rot * jnp.maximum(accs[c] + b1rot, 0.0)
        out_v[pl.ds(g * 16, 16)] = 1.0 / (1.0 + jnp.exp(-z))
        return carry

    lax.fori_loop(0, GROUPS, group_body, 0)
    pltpu.sync_copy(out_v, out_hbm.at[pl.ds(base, BAGS_PER_W)])


_sc_call = functools.partial(
    pl.kernel,
    out_type=jax.ShapeDtypeStruct((B,), jnp.float32),
    mesh=plsc.VectorSubcoreMesh(core_axis_name="c", subcore_axis_name="s"),
    compiler_params=pltpu.CompilerParams(
        use_tc_tiling_on_sc=False,
        needs_layout_passes=False,
        disable_bounds_checks=True,
    ),
    scratch_types=[
        pltpu.VMEM((VPAD, H1), jnp.float32),
        pltpu.VMEM((BAGS_PER_W, LPAD), jnp.int32),
        pltpu.VMEM((BAGS_PER_W,), jnp.float32),
        pltpu.VMEM((48,), jnp.float32),
    ],
)(_sc_body)


def kernel(x, table, W1, b1, W2, b2):
    table_pad = jnp.pad(table, ((0, VPAD - VOCAB), (0, 0)))
    ptable = _make_ptable(table_pad, W1.T)
    wpack = jnp.concatenate([
        jnp.zeros((8,), jnp.float32), b1, W2.reshape(H1),
        jnp.broadcast_to(b2, (8,)),
    ]).astype(jnp.float32)
    out = _sc_call(ptable, x, wpack)
    return out.reshape(B, 1)
